# baseline (speedup 1.0000x reference)
import functools

import jax
import jax.numpy as jnp
from jax import lax
from jax.experimental import pallas as pl
from jax.experimental.pallas import tpu as pltpu
from jax.experimental.pallas import tpu_sc as plsc

EMBED_DIM = 32
BATCH = 16384
LANES = 16
NUM_CORES = 2
NUM_SUBCORES = 16
NUM_WORKERS = NUM_CORES * NUM_SUBCORES
B_PER_W = BATCH // NUM_WORKERS
GROUPS = B_PER_W // LANES
CHUNK = 128
N_CHUNKS = B_PER_W // CHUNK

_mesh = plsc.VectorSubcoreMesh(core_axis_name="c", subcore_axis_name="s")


@functools.partial(
    pl.kernel,
    mesh=_mesh,
    compiler_params=pltpu.CompilerParams(
        needs_layout_passes=False, use_tc_tiling_on_sc=False),
    out_type=jax.ShapeDtypeStruct((BATCH,), jnp.float32),
    scratch_types=[
        pltpu.VMEM((N_CHUNKS, CHUNK), jnp.int32),
        pltpu.VMEM((N_CHUNKS, CHUNK), jnp.int32),
        pltpu.VMEM((B_PER_W, EMBED_DIM), jnp.float32),
        pltpu.VMEM((B_PER_W, EMBED_DIM), jnp.float32),
        pltpu.VMEM((B_PER_W,), jnp.float32),
        pltpu.VMEM((B_PER_W,), jnp.float32),
        pltpu.VMEM((B_PER_W,), jnp.float32),
        pltpu.SemaphoreType.DMA,
        pltpu.SemaphoreType.DMA,
        pltpu.SemaphoreType.DMA,
        pltpu.SemaphoreType.DMA,
    ],
)
def _bilinear_sc(uid_hbm, iid_hbm, uemb_hbm, iemb_hbm, ubias_hbm, ibias_hbm,
                 out_hbm, uidx_v, iidx_v, urows_v, irows_v, ub_v, ib_v,
                 out_v, sem_u, sem_i, sem_ub, sem_ib):
    c = lax.axis_index("c")
    s = lax.axis_index("s")
    wid = s * NUM_CORES + c
    base = pl.multiple_of(wid * B_PER_W, B_PER_W)

    pltpu.sync_copy(uid_hbm.at[wid], uidx_v)
    pltpu.sync_copy(iid_hbm.at[wid], iidx_v)

    copies = []
    for k in range(N_CHUNKS):
        sl = pl.ds(k * CHUNK, CHUNK)
        copies.append(
            pltpu.async_copy(uemb_hbm.at[uidx_v.at[k]], urows_v.at[sl], sem_u))
        copies.append(
            pltpu.async_copy(iemb_hbm.at[iidx_v.at[k]], irows_v.at[sl], sem_i))
        copies.append(
            pltpu.async_copy(ubias_hbm.at[uidx_v.at[k]], ub_v.at[sl], sem_ub))
        copies.append(
            pltpu.async_copy(ibias_hbm.at[iidx_v.at[k]], ib_v.at[sl], sem_ib))
    for cp in copies:
        cp.wait()

    lane = lax.iota(jnp.int32, LANES)

    def group_body(g, carry):
        gbase = pl.multiple_of(g * LANES, LANES)
        rows = gbase + lane
        acc = ub_v[pl.ds(gbase, LANES)] + ib_v[pl.ds(gbase, LANES)]
        for d in range(EMBED_DIM):
            dcol = jnp.full((LANES,), d, jnp.int32)
            uv = plsc.load_gather(urows_v, [rows, dcol])
            iv = plsc.load_gather(irows_v, [rows, dcol])
            acc = acc + uv * iv
        out_v[pl.ds(gbase, LANES)] = acc
        return carry

    lax.fori_loop(0, GROUPS, group_body, 0)

    pltpu.sync_copy(out_v, out_hbm.at[pl.ds(base, B_PER_W)])


def kernel(user_ids, item_ids, user_emb, item_emb, user_bias, item_bias):
    uids = user_ids.astype(jnp.int32).reshape(NUM_WORKERS, N_CHUNKS, CHUNK)
    iids = item_ids.astype(jnp.int32).reshape(NUM_WORKERS, N_CHUNKS, CHUNK)
    ub = user_bias.reshape(-1)
    ib = item_bias.reshape(-1)
    return _bilinear_sc(uids, iids, user_emb, item_emb, ub, ib)
